# baseline (speedup 1.0000x reference)
import functools

import jax
import jax.numpy as jnp
from jax import lax
from jax.experimental import pallas as pl
from jax.experimental.pallas import tpu as pltpu
from jax.experimental.pallas import tpu_sc as plsc

N = 10000
E = 320000
DF = 128
H = 32
LAT = 16

NC = 2
NS = 16
NW = NC * NS
CH = 125
CHUNKS = E // CH
TCH = CHUNKS // NW
NB = 8
CP_TILES = 10
ROWS_PT = N // CP_TILES

_mesh = plsc.VectorSubcoreMesh(core_axis_name="c", subcore_axis_name="s")


@functools.partial(
    pl.kernel,
    out_type=jax.ShapeDtypeStruct((NC, N, 8), jnp.float32),
    mesh=_mesh,
    compiler_params=pltpu.CompilerParams(use_tc_tiling_on_sc=False),
    scratch_types=[
        pltpu.VMEM_SHARED((N, 8), jnp.float32),
        pltpu.VMEM((TCH, CH), jnp.int32),
        pltpu.VMEM((CH, 8), jnp.float32),
        pltpu.SemaphoreType.DMA,
    ],
)
def _deg_kernel(dst2, ones, zeros8, out, deg_sh, idx_d, obuf, sem):
    cid = lax.axis_index("c")
    sid = lax.axis_index("s")
    wid = sid * NC + cid

    @pl.when(sid < CP_TILES)
    def _():
        r0 = sid * ROWS_PT
        pltpu.sync_copy(zeros8.at[pl.ds(r0, ROWS_PT)],
                        deg_sh.at[pl.ds(r0, ROWS_PT)])

    pltpu.sync_copy(dst2.at[pl.ds(wid * TCH, TCH)], idx_d)
    pltpu.sync_copy(ones, obuf)
    plsc.subcore_barrier()

    def step(j0, carry):
        descs = []
        for b in range(8):
            j = j0 * 8 + b
            descs.append(
                pltpu.async_copy(obuf, deg_sh.at[idx_d.at[j]], sem, add=True))
        for d in descs:
            d.wait()
        return carry

    lax.fori_loop(0, TCH // 8, step, 0)
    plsc.subcore_barrier()

    @pl.when(sid < CP_TILES)
    def _():
        r0 = sid * ROWS_PT
        pltpu.sync_copy(deg_sh.at[pl.ds(r0, ROWS_PT)],
                        out.at[cid, pl.ds(r0, ROWS_PT)])


@functools.partial(
    pl.kernel,
    out_type=jax.ShapeDtypeStruct((NC, N, H), jnp.float32),
    mesh=_mesh,
    compiler_params=pltpu.CompilerParams(use_tc_tiling_on_sc=False),
    scratch_types=[
        pltpu.VMEM_SHARED((N, H), jnp.float32),
        pltpu.VMEM_SHARED((N, H), jnp.float32),
        pltpu.VMEM((TCH, CH), jnp.int32),
        pltpu.VMEM((TCH, CH), jnp.int32),
        pltpu.VMEM((NB, CH, H), jnp.float32),
    ] + [pltpu.SemaphoreType.DMA] * NB,
)
def _prop_kernel(table, src2, dst2, zeros, out, acc_sh, tab_sh, idx_s, idx_d,
                 rows, *gsems):
    cid = lax.axis_index("c")
    sid = lax.axis_index("s")
    wid = sid * NC + cid

    @pl.when(sid < CP_TILES)
    def _():
        r0 = sid * ROWS_PT
        pltpu.sync_copy(zeros.at[pl.ds(r0, ROWS_PT)],
                        acc_sh.at[pl.ds(r0, ROWS_PT)])
        pltpu.sync_copy(table.at[pl.ds(r0, ROWS_PT)],
                        tab_sh.at[pl.ds(r0, ROWS_PT)])

    base = wid * TCH
    pltpu.sync_copy(src2.at[pl.ds(base, TCH)], idx_s)
    pltpu.sync_copy(dst2.at[pl.ds(base, TCH)], idx_d)
    plsc.subcore_barrier()

    for b in range(NB):
        pltpu.async_copy(tab_sh.at[idx_s.at[b]], rows.at[b], gsems[b])

    def step(j0, carry):
        for b in range(NB):
            j = j0 * NB + b
            pltpu.make_async_copy(tab_sh.at[idx_s.at[j]], rows.at[b],
                                  gsems[b]).wait()
            pltpu.sync_copy(rows.at[b], acc_sh.at[idx_d.at[j]], add=True)
            pltpu.async_copy(tab_sh.at[idx_s.at[j + NB]], rows.at[b], gsems[b])
        return carry

    lax.fori_loop(0, TCH // NB - 1, step, 0)

    for b in range(NB):
        j = TCH - NB + b
        pltpu.make_async_copy(tab_sh.at[idx_s.at[j]], rows.at[b],
                              gsems[b]).wait()
        pltpu.sync_copy(rows.at[b], acc_sh.at[idx_d.at[j]], add=True)

    plsc.subcore_barrier()

    @pl.when(sid < CP_TILES)
    def _():
        r0 = sid * ROWS_PT
        pltpu.sync_copy(acc_sh.at[pl.ds(r0, ROWS_PT)],
                        out.at[cid, pl.ds(r0, ROWS_PT)])


_GRID = 10
_BR = N // _GRID


def _dis(degp_ref):
    deg = degp_ref[0][:, 0:1] + degp_ref[1][:, 0:1] + 1.0
    return lax.rsqrt(deg)


def _tc1_body(x_ref, w1_ref, degp_ref, g2_ref):
    g = jnp.dot(x_ref[...], w1_ref[...], preferred_element_type=jnp.float32)
    g2_ref[...] = g * _dis(degp_ref)


_tc1 = pl.pallas_call(
    _tc1_body,
    grid=(_GRID,),
    in_specs=[
        pl.BlockSpec((_BR, DF), lambda i: (i, 0)),
        pl.BlockSpec((DF, H), lambda i: (0, 0)),
        pl.BlockSpec((NC, _BR, 8), lambda i: (0, i, 0)),
    ],
    out_specs=pl.BlockSpec((_BR, H), lambda i: (i, 0)),
    out_shape=jax.ShapeDtypeStruct((N, H), jnp.float32),
)


def _tc2_body(g2_ref, acc_ref, degp_ref, b1_ref, h2_ref):
    dis = _dis(degp_ref)
    s = dis * (g2_ref[...] + acc_ref[0] + acc_ref[1]) + b1_ref[...]
    h2_ref[...] = dis * jnp.maximum(s, 0.0)


_tc2 = pl.pallas_call(
    _tc2_body,
    grid=(_GRID,),
    in_specs=[
        pl.BlockSpec((_BR, H), lambda i: (i, 0)),
        pl.BlockSpec((NC, _BR, H), lambda i: (0, i, 0)),
        pl.BlockSpec((NC, _BR, 8), lambda i: (0, i, 0)),
        pl.BlockSpec((1, H), lambda i: (0, 0)),
    ],
    out_specs=pl.BlockSpec((_BR, H), lambda i: (i, 0)),
    out_shape=jax.ShapeDtypeStruct((N, H), jnp.float32),
)


def _tc3_body(h2_ref, acc_ref, degp_ref, wm_ref, bm_ref, wv_ref, bv_ref,
              mean_ref, var_ref):
    dis = _dis(degp_ref)
    p = dis * (h2_ref[...] + acc_ref[0] + acc_ref[1])
    mean_ref[...] = jnp.dot(p, wm_ref[...],
                            preferred_element_type=jnp.float32) + bm_ref[...]
    var_ref[...] = jnp.dot(p, wv_ref[...],
                           preferred_element_type=jnp.float32) + bv_ref[...]


_tc3 = pl.pallas_call(
    _tc3_body,
    grid=(_GRID,),
    in_specs=[
        pl.BlockSpec((_BR, H), lambda i: (i, 0)),
        pl.BlockSpec((NC, _BR, H), lambda i: (0, i, 0)),
        pl.BlockSpec((NC, _BR, 8), lambda i: (0, i, 0)),
        pl.BlockSpec((H, LAT), lambda i: (0, 0)),
        pl.BlockSpec((1, LAT), lambda i: (0, 0)),
        pl.BlockSpec((H, LAT), lambda i: (0, 0)),
        pl.BlockSpec((1, LAT), lambda i: (0, 0)),
    ],
    out_specs=[
        pl.BlockSpec((_BR, LAT), lambda i: (i, 0)),
        pl.BlockSpec((_BR, LAT), lambda i: (i, 0)),
    ],
    out_shape=[
        jax.ShapeDtypeStruct((N, LAT), jnp.float32),
        jax.ShapeDtypeStruct((N, LAT), jnp.float32),
    ],
)


@jax.jit
def kernel(features, edge_index, W1, b1, Wm, bm, Wv, bv):
    src2 = edge_index[0].reshape(CHUNKS, CH)
    dst2 = edge_index[1].reshape(CHUNKS, CH)
    zeros = jnp.zeros((N, H), jnp.float32)
    zeros8 = jnp.zeros((N, 8), jnp.float32)
    ones = jnp.ones((CH, 8), jnp.float32)

    degp = _deg_kernel(dst2, ones, zeros8)
    g2 = _tc1(features, W1, degp)
    acc1 = _prop_kernel(g2, src2, dst2, zeros)
    h2 = _tc2(g2, acc1, degp, b1.reshape(1, H))
    acc2 = _prop_kernel(h2, src2, dst2, zeros)
    mean, var = _tc3(h2, acc2, degp, Wm, bm.reshape(1, LAT),
                     Wv, bv.reshape(1, LAT))
    return (mean, var)


# speedup vs baseline: 1.0194x; 1.0194x over previous
import functools

import jax
import jax.numpy as jnp
from jax import lax
from jax.experimental import pallas as pl
from jax.experimental.pallas import tpu as pltpu
from jax.experimental.pallas import tpu_sc as plsc

N = 10000
E = 320000
DF = 128
H = 32
LAT = 16

NC = 2
NS = 16
NW = NC * NS
CH = 125
CHUNKS = E // CH
TCH = CHUNKS // NW
NB = 8
CP_TILES = 10
ROWS_PT = N // CP_TILES
RT = N // NS
RTP = 640
NG = RTP // 16

_mesh = plsc.VectorSubcoreMesh(core_axis_name="c", subcore_axis_name="s")
_sc_params = pltpu.CompilerParams(use_tc_tiling_on_sc=False)


def _rsqrt16(x):
    y = jnp.full((16,), 2.0 ** -10, jnp.float32)
    for k in range(9, 0, -1):
        y = jnp.where(x < float(4 ** k), jnp.float32(2.0 ** -k), y)
    for _ in range(6):
        y = y * (1.5 - 0.5 * x * y * y)
    return y


def _iota16():
    return lax.iota(jnp.int32, 16)


@functools.partial(
    pl.kernel,
    out_type=jax.ShapeDtypeStruct((NC, N, 8), jnp.float32),
    mesh=_mesh,
    compiler_params=_sc_params,
    scratch_types=[
        pltpu.VMEM_SHARED((N, 8), jnp.float32),
        pltpu.VMEM((TCH, CH), jnp.int32),
        pltpu.VMEM((CH, 8), jnp.float32),
        pltpu.SemaphoreType.DMA,
    ],
)
def _deg_kernel(dst2, ones, zeros8, out, deg_sh, idx_d, obuf, sem):
    cid = lax.axis_index("c")
    sid = lax.axis_index("s")
    wid = sid * NC + cid

    @pl.when(sid < CP_TILES)
    def _():
        r0 = sid * ROWS_PT
        pltpu.sync_copy(zeros8.at[pl.ds(r0, ROWS_PT)],
                        deg_sh.at[pl.ds(r0, ROWS_PT)])

    pltpu.sync_copy(dst2.at[pl.ds(wid * TCH, TCH)], idx_d)
    pltpu.sync_copy(ones, obuf)
    plsc.subcore_barrier()

    def step(j0, carry):
        descs = []
        for b in range(8):
            j = j0 * 8 + b
            descs.append(
                pltpu.async_copy(obuf, deg_sh.at[idx_d.at[j]], sem, add=True))
        for d in descs:
            d.wait()
        return carry

    lax.fori_loop(0, TCH // 8, step, 0)
    plsc.subcore_barrier()

    @pl.when(sid < CP_TILES)
    def _():
        r0 = sid * ROWS_PT
        pltpu.sync_copy(deg_sh.at[pl.ds(r0, ROWS_PT)],
                        out.at[cid, pl.ds(r0, ROWS_PT)])


def _edge_pass(tabref, acc_sh, idx_s, idx_d, rows, gsems):
    for b in range(NB):
        pltpu.async_copy(tabref.at[idx_s.at[b]], rows.at[b], gsems[b])

    def step(j0, carry):
        for b in range(NB):
            j = j0 * NB + b
            pltpu.make_async_copy(tabref.at[idx_s.at[j]], rows.at[b],
                                  gsems[b]).wait()
            pltpu.sync_copy(rows.at[b], acc_sh.at[idx_d.at[j]], add=True)
            pltpu.async_copy(tabref.at[idx_s.at[j + NB]], rows.at[b], gsems[b])
        return carry

    lax.fori_loop(0, TCH // NB - 1, step, 0)

    for b in range(NB):
        j = TCH - NB + b
        pltpu.make_async_copy(tabref.at[idx_s.at[j]], rows.at[b],
                              gsems[b]).wait()
        pltpu.sync_copy(rows.at[b], acc_sh.at[idx_d.at[j]], add=True)


def _load_edge_indices(src2, dst2, idx_s, idx_d, wid):
    base = wid * TCH
    pltpu.sync_copy(src2.at[pl.ds(base, TCH)], idx_s)
    pltpu.sync_copy(dst2.at[pl.ds(base, TCH)], idx_d)


def _zero_acc(acc_sh, rows, sid):
    @pl.when(sid < CP_TILES)
    def _():
        z16 = jnp.zeros((16,), jnp.float32)

        def zrow(e, carry):
            rows[0, e, pl.ds(0, 16)] = z16
            rows[0, e, pl.ds(16, 16)] = z16
            return carry

        lax.fori_loop(0, CH, zrow, 0)
        r0 = sid * ROWS_PT
        for q in range(ROWS_PT // CH):
            pltpu.sync_copy(rows.at[0], acc_sh.at[pl.ds(r0 + q * CH, CH)])


def _copy_out_acc(acc_sh, out, cid, sid):
    @pl.when(sid < CP_TILES)
    def _():
        r0 = sid * ROWS_PT
        pltpu.sync_copy(acc_sh.at[pl.ds(r0, ROWS_PT)],
                        out.at[cid, pl.ds(r0, ROWS_PT)])


@functools.partial(
    pl.kernel,
    out_type=(
        jax.ShapeDtypeStruct((N, H), jnp.float32),
        jax.ShapeDtypeStruct((NC, N, H), jnp.float32),
    ),
    mesh=_mesh,
    compiler_params=_sc_params,
    scratch_types=[
        pltpu.VMEM_SHARED((N, H), jnp.float32),
        pltpu.VMEM((TCH, CH), jnp.int32),
        pltpu.VMEM((TCH, CH), jnp.int32),
        pltpu.VMEM((NB, CH, H), jnp.float32),
        pltpu.VMEM((RTP, H), jnp.float32),
        pltpu.VMEM((RTP * 8,), jnp.float32),
        pltpu.VMEM((RTP * 8,), jnp.float32),
    ] + [pltpu.SemaphoreType.DMA] * NB,
)
def _sca_kernel(g, src2, dst2, degpf, gs, out, acc_sh, idx_s, idx_d,
                rows, gbuf, dbufa, dbufb, *gsems):
    cid = lax.axis_index("c")
    sid = lax.axis_index("s")
    wid = sid * NC + cid

    _zero_acc(acc_sh, rows, sid)
    _load_edge_indices(src2, dst2, idx_s, idx_d, wid)

    r0 = sid * RT
    pltpu.sync_copy(g.at[pl.ds(r0, RT)], gbuf.at[pl.ds(0, RT)])
    pltpu.sync_copy(degpf.at[0, pl.ds(r0 * 8, RT * 8)],
                    dbufa.at[pl.ds(0, RT * 8)])
    pltpu.sync_copy(degpf.at[1, pl.ds(r0 * 8, RT * 8)],
                    dbufb.at[pl.ds(0, RT * 8)])

    def scale_pair(k, carry):
        off = k * 16
        v = dbufa[pl.ds(off, 16)] + dbufb[pl.ds(off, 16)] + 1.0
        d16 = _rsqrt16(v)
        for j in range(2):
            sdis = d16[j * 8]
            r = k * 2 + j
            for half in range(2):
                sl = pl.ds(half * 16, 16)
                gbuf[r, sl] = gbuf[r, sl] * sdis
        return carry

    lax.fori_loop(0, RT // 2, scale_pair, 0)
    v = dbufa[pl.ds((RT - 1) * 8, 16)] + dbufb[pl.ds((RT - 1) * 8, 16)] + 1.0
    sdis = _rsqrt16(v)[0]
    for half in range(2):
        sl = pl.ds(half * 16, 16)
        gbuf[RT - 1, sl] = gbuf[RT - 1, sl] * sdis
    pltpu.sync_copy(gbuf.at[pl.ds(0, RT)], gs.at[pl.ds(r0, RT)])
    plsc.subcore_barrier()

    _edge_pass(gs, acc_sh, idx_s, idx_d, rows, gsems)
    plsc.subcore_barrier()
    _copy_out_acc(acc_sh, out, cid, sid)


@functools.partial(
    pl.kernel,
    out_type=(
        jax.ShapeDtypeStruct((N, H), jnp.float32),
        jax.ShapeDtypeStruct((NC, N, H), jnp.float32),
    ),
    mesh=_mesh,
    compiler_params=_sc_params,
    scratch_types=[
        pltpu.VMEM_SHARED((N, H), jnp.float32),
        pltpu.VMEM((TCH, CH), jnp.int32),
        pltpu.VMEM((TCH, CH), jnp.int32),
        pltpu.VMEM((NB, CH, H), jnp.float32),
        pltpu.VMEM((RTP, H), jnp.float32),
        pltpu.VMEM((RTP, H), jnp.float32),
        pltpu.VMEM((RTP * 8,), jnp.float32),
        pltpu.VMEM((RTP * 8,), jnp.float32),
        pltpu.VMEM((1, H), jnp.float32),
    ] + [pltpu.SemaphoreType.DMA] * NB,
)
def _scb_kernel(gs, acc1, degpf, b1, src2, dst2, h2x, out, acc_sh,
                idx_s, idx_d, rows, gbuf, abuf, dbufa, dbufb, b1buf,
                *gsems):
    cid = lax.axis_index("c")
    sid = lax.axis_index("s")
    wid = sid * NC + cid

    _zero_acc(acc_sh, rows, sid)
    _load_edge_indices(src2, dst2, idx_s, idx_d, wid)

    r0 = sid * RT
    pltpu.sync_copy(gs.at[pl.ds(r0, RT)], gbuf.at[pl.ds(0, RT)])
    pltpu.sync_copy(acc1.at[0, pl.ds(r0, RT)], abuf.at[pl.ds(0, RT)])
    pltpu.sync_copy(degpf.at[0, pl.ds(r0 * 8, RT * 8)],
                    dbufa.at[pl.ds(0, RT * 8)])
    pltpu.sync_copy(degpf.at[1, pl.ds(r0 * 8, RT * 8)],
                    dbufb.at[pl.ds(0, RT * 8)])
    pltpu.sync_copy(b1, b1buf)

    for q in range(RT // CH):
        pltpu.sync_copy(acc1.at[1, pl.ds(r0 + q * CH, CH)], rows.at[q])

        def add_row(e, carry, q=q):
            for half in range(2):
                sl = pl.ds(half * 16, 16)
                abuf[q * CH + e, sl] = abuf[q * CH + e, sl] + rows[q, e, sl]
            return carry

        lax.fori_loop(0, CH, add_row, 0)

    b1lo = b1buf[0, pl.ds(0, 16)]
    b1hi = b1buf[0, pl.ds(16, 16)]

    def h2_row(r, sdis):
        for half in range(2):
            sl = pl.ds(half * 16, 16)
            z = sdis * (gbuf[r, sl] + abuf[r, sl])
            z = z + (b1lo if half == 0 else b1hi)
            gbuf[r, sl] = sdis * jnp.maximum(z, 0.0)

    def h2_pair(k, carry):
        off = k * 16
        v = dbufa[pl.ds(off, 16)] + dbufb[pl.ds(off, 16)] + 1.0
        d16 = _rsqrt16(v)
        for j in range(2):
            h2_row(k * 2 + j, d16[j * 8])
        return carry

    lax.fori_loop(0, RT // 2, h2_pair, 0)
    v = dbufa[pl.ds((RT - 1) * 8, 16)] + dbufb[pl.ds((RT - 1) * 8, 16)] + 1.0
    h2_row(RT - 1, _rsqrt16(v)[0])
    pltpu.sync_copy(gbuf.at[pl.ds(0, RT)], h2x.at[pl.ds(r0, RT)])
    plsc.subcore_barrier()

    _edge_pass(h2x, acc_sh, idx_s, idx_d, rows, gsems)
    plsc.subcore_barrier()
    _copy_out_acc(acc_sh, out, cid, sid)


_GRID = 10
_BR = N // _GRID


def _tc1_body(x_ref, w1_ref, g_ref):
    g_ref[...] = jnp.dot(x_ref[...], w1_ref[...],
                         preferred_element_type=jnp.float32)


_tc1 = pl.pallas_call(
    _tc1_body,
    grid=(_GRID,),
    in_specs=[
        pl.BlockSpec((_BR, DF), lambda i: (i, 0)),
        pl.BlockSpec((DF, H), lambda i: (0, 0)),
    ],
    out_specs=pl.BlockSpec((_BR, H), lambda i: (i, 0)),
    out_shape=jax.ShapeDtypeStruct((N, H), jnp.float32),
)


def _tc3_body(h2_ref, acc_ref, degp_ref, wm_ref, bm_ref, wv_ref, bv_ref,
              mean_ref, var_ref):
    deg = degp_ref[0][:, 0:1] + degp_ref[1][:, 0:1] + 1.0
    dis = lax.rsqrt(deg)
    p = dis * (h2_ref[...] + acc_ref[0] + acc_ref[1])
    mean_ref[...] = jnp.dot(p, wm_ref[...],
                            preferred_element_type=jnp.float32) + bm_ref[...]
    var_ref[...] = jnp.dot(p, wv_ref[...],
                           preferred_element_type=jnp.float32) + bv_ref[...]


_tc3 = pl.pallas_call(
    _tc3_body,
    grid=(_GRID,),
    in_specs=[
        pl.BlockSpec((_BR, H), lambda i: (i, 0)),
        pl.BlockSpec((NC, _BR, H), lambda i: (0, i, 0)),
        pl.BlockSpec((NC, _BR, 8), lambda i: (0, i, 0)),
        pl.BlockSpec((H, LAT), lambda i: (0, 0)),
        pl.BlockSpec((1, LAT), lambda i: (0, 0)),
        pl.BlockSpec((H, LAT), lambda i: (0, 0)),
        pl.BlockSpec((1, LAT), lambda i: (0, 0)),
    ],
    out_specs=[
        pl.BlockSpec((_BR, LAT), lambda i: (i, 0)),
        pl.BlockSpec((_BR, LAT), lambda i: (i, 0)),
    ],
    out_shape=[
        jax.ShapeDtypeStruct((N, LAT), jnp.float32),
        jax.ShapeDtypeStruct((N, LAT), jnp.float32),
    ],
)


@jax.jit
def kernel(features, edge_index, W1, b1, Wm, bm, Wv, bv):
    src2 = edge_index[0].reshape(CHUNKS, CH)
    dst2 = edge_index[1].reshape(CHUNKS, CH)
    zeros8 = jnp.zeros((N, 8), jnp.float32)
    ones = jnp.ones((CH, 8), jnp.float32)

    degp = _deg_kernel(dst2, ones, zeros8)
    degpf = degp.reshape(NC, N * 8)
    g = _tc1(features, W1)
    gs, acc1 = _sca_kernel(g, src2, dst2, degpf)
    h2x, acc2 = _scb_kernel(gs, acc1, degpf, b1.reshape(1, H),
                            src2, dst2)
    mean, var = _tc3(h2x, acc2, degp, Wm, bm.reshape(1, LAT),
                     Wv, bv.reshape(1, LAT))
    return (mean, var)


# speedup vs baseline: 1.1549x; 1.1329x over previous
import functools

import jax
import jax.numpy as jnp
from jax import lax
from jax.experimental import pallas as pl
from jax.experimental.pallas import tpu as pltpu
from jax.experimental.pallas import tpu_sc as plsc

N = 10000
E = 320000
DF = 128
H = 32
LAT = 16

NC = 2
NS = 16
NW = NC * NS
CH = 125
CHUNKS = E // CH
TCH = CHUNKS // NW
NB = 10
CP_TILES = 10
ROWS_PT = N // CP_TILES

_mesh = plsc.VectorSubcoreMesh(core_axis_name="c", subcore_axis_name="s")


@functools.partial(
    pl.kernel,
    out_type=jax.ShapeDtypeStruct((NC, N, 8), jnp.float32),
    mesh=_mesh,
    compiler_params=pltpu.CompilerParams(use_tc_tiling_on_sc=False),
    scratch_types=[
        pltpu.VMEM_SHARED((N, 8), jnp.float32),
        pltpu.VMEM((TCH, CH), jnp.int32),
        pltpu.VMEM((CH, 8), jnp.float32),
        pltpu.SemaphoreType.DMA,
    ],
)
def _deg_kernel(dst2, ones, zeros8, out, deg_sh, idx_d, obuf, sem):
    cid = lax.axis_index("c")
    sid = lax.axis_index("s")
    wid = sid * NC + cid

    @pl.when(sid < CP_TILES)
    def _():
        r0 = sid * ROWS_PT
        pltpu.sync_copy(zeros8.at[pl.ds(r0, ROWS_PT)],
                        deg_sh.at[pl.ds(r0, ROWS_PT)])

    pltpu.sync_copy(dst2.at[pl.ds(wid * TCH, TCH)], idx_d)
    pltpu.sync_copy(ones, obuf)
    plsc.subcore_barrier()

    def step(j0, carry):
        descs = []
        for b in range(8):
            j = j0 * 8 + b
            descs.append(
                pltpu.async_copy(obuf, deg_sh.at[idx_d.at[j]], sem, add=True))
        for d in descs:
            d.wait()
        return carry

    lax.fori_loop(0, TCH // 8, step, 0)
    plsc.subcore_barrier()

    @pl.when(sid < CP_TILES)
    def _():
        r0 = sid * ROWS_PT
        pltpu.sync_copy(deg_sh.at[pl.ds(r0, ROWS_PT)],
                        out.at[cid, pl.ds(r0, ROWS_PT)])


@functools.partial(
    pl.kernel,
    out_type=jax.ShapeDtypeStruct((NC, N, H), jnp.float32),
    mesh=_mesh,
    compiler_params=pltpu.CompilerParams(use_tc_tiling_on_sc=False),
    scratch_types=[
        pltpu.VMEM_SHARED((N, H), jnp.float32),
        pltpu.VMEM((TCH, CH), jnp.int32),
        pltpu.VMEM((TCH, CH), jnp.int32),
        pltpu.VMEM((NB, CH, H), jnp.float32),
    ] + [pltpu.SemaphoreType.DMA] * NB,
)
def _prop_kernel(table, src2, dst2, zeros, out, acc_sh, idx_s, idx_d,
                 rows, *gsems):
    cid = lax.axis_index("c")
    sid = lax.axis_index("s")
    wid = sid * NC + cid

    @pl.when(sid < CP_TILES)
    def _():
        r0 = sid * ROWS_PT
        pltpu.sync_copy(zeros.at[pl.ds(r0, ROWS_PT)],
                        acc_sh.at[pl.ds(r0, ROWS_PT)])

    base = wid * TCH
    pltpu.sync_copy(src2.at[pl.ds(base, TCH)], idx_s)
    pltpu.sync_copy(dst2.at[pl.ds(base, TCH)], idx_d)
    plsc.subcore_barrier()

    for b in range(NB):
        pltpu.async_copy(table.at[idx_s.at[b]], rows.at[b], gsems[b])

    def step(j0, carry):
        for b in range(NB):
            j = j0 * NB + b
            pltpu.make_async_copy(table.at[idx_s.at[j]], rows.at[b],
                                  gsems[b]).wait()
            pltpu.sync_copy(rows.at[b], acc_sh.at[idx_d.at[j]], add=True)
            pltpu.async_copy(table.at[idx_s.at[j + NB]], rows.at[b], gsems[b])
        return carry

    lax.fori_loop(0, TCH // NB - 1, step, 0)

    for b in range(NB):
        j = TCH - NB + b
        pltpu.make_async_copy(table.at[idx_s.at[j]], rows.at[b],
                              gsems[b]).wait()
        pltpu.sync_copy(rows.at[b], acc_sh.at[idx_d.at[j]], add=True)

    plsc.subcore_barrier()

    @pl.when(sid < CP_TILES)
    def _():
        r0 = sid * ROWS_PT
        pltpu.sync_copy(acc_sh.at[pl.ds(r0, ROWS_PT)],
                        out.at[cid, pl.ds(r0, ROWS_PT)])


_GRID = 5
_BR = N // _GRID


def _dis(degp_ref):
    deg = degp_ref[0][:, 0:1] + degp_ref[1][:, 0:1] + 1.0
    return lax.rsqrt(deg)


def _tc1_body(x_ref, w1_ref, degp_ref, g2_ref):
    g = jnp.dot(x_ref[...], w1_ref[...], preferred_element_type=jnp.float32)
    g2_ref[...] = g * _dis(degp_ref)


_tc1 = pl.pallas_call(
    _tc1_body,
    grid=(_GRID,),
    in_specs=[
        pl.BlockSpec((_BR, DF), lambda i: (i, 0)),
        pl.BlockSpec((DF, H), lambda i: (0, 0)),
        pl.BlockSpec((NC, _BR, 8), lambda i: (0, i, 0)),
    ],
    out_specs=pl.BlockSpec((_BR, H), lambda i: (i, 0)),
    out_shape=jax.ShapeDtypeStruct((N, H), jnp.float32),
)


def _tc2_body(g2_ref, acc_ref, degp_ref, b1_ref, h2_ref):
    dis = _dis(degp_ref)
    s = dis * (g2_ref[...] + acc_ref[0] + acc_ref[1]) + b1_ref[...]
    h2_ref[...] = dis * jnp.maximum(s, 0.0)


_tc2 = pl.pallas_call(
    _tc2_body,
    grid=(_GRID,),
    in_specs=[
        pl.BlockSpec((_BR, H), lambda i: (i, 0)),
        pl.BlockSpec((NC, _BR, H), lambda i: (0, i, 0)),
        pl.BlockSpec((NC, _BR, 8), lambda i: (0, i, 0)),
        pl.BlockSpec((1, H), lambda i: (0, 0)),
    ],
    out_specs=pl.BlockSpec((_BR, H), lambda i: (i, 0)),
    out_shape=jax.ShapeDtypeStruct((N, H), jnp.float32),
)


def _tc3_body(h2_ref, acc_ref, degp_ref, wm_ref, bm_ref, wv_ref, bv_ref,
              mean_ref, var_ref):
    dis = _dis(degp_ref)
    p = dis * (h2_ref[...] + acc_ref[0] + acc_ref[1])
    mean_ref[...] = jnp.dot(p, wm_ref[...],
                            preferred_element_type=jnp.float32) + bm_ref[...]
    var_ref[...] = jnp.dot(p, wv_ref[...],
                           preferred_element_type=jnp.float32) + bv_ref[...]


_tc3 = pl.pallas_call(
    _tc3_body,
    grid=(_GRID,),
    in_specs=[
        pl.BlockSpec((_BR, H), lambda i: (i, 0)),
        pl.BlockSpec((NC, _BR, H), lambda i: (0, i, 0)),
        pl.BlockSpec((NC, _BR, 8), lambda i: (0, i, 0)),
        pl.BlockSpec((H, LAT), lambda i: (0, 0)),
        pl.BlockSpec((1, LAT), lambda i: (0, 0)),
        pl.BlockSpec((H, LAT), lambda i: (0, 0)),
        pl.BlockSpec((1, LAT), lambda i: (0, 0)),
    ],
    out_specs=[
        pl.BlockSpec((_BR, LAT), lambda i: (i, 0)),
        pl.BlockSpec((_BR, LAT), lambda i: (i, 0)),
    ],
    out_shape=[
        jax.ShapeDtypeStruct((N, LAT), jnp.float32),
        jax.ShapeDtypeStruct((N, LAT), jnp.float32),
    ],
)


@jax.jit
def kernel(features, edge_index, W1, b1, Wm, bm, Wv, bv):
    src2 = edge_index[0].reshape(CHUNKS, CH)
    dst2 = edge_index[1].reshape(CHUNKS, CH)
    zeros = jnp.zeros((N, H), jnp.float32)
    zeros8 = jnp.zeros((N, 8), jnp.float32)
    ones = jnp.ones((CH, 8), jnp.float32)

    degp = _deg_kernel(dst2, ones, zeros8)
    g2 = _tc1(features, W1, degp)
    acc1 = _prop_kernel(g2, src2, dst2, zeros)
    h2 = _tc2(g2, acc1, degp, b1.reshape(1, H))
    acc2 = _prop_kernel(h2, src2, dst2, zeros)
    mean, var = _tc3(h2, acc2, degp, Wm, bm.reshape(1, LAT),
                     Wv, bv.reshape(1, LAT))
    return (mean, var)


# speedup vs baseline: 1.3455x; 1.1651x over previous
import functools

import jax
import jax.numpy as jnp
from jax import lax
from jax.experimental import pallas as pl
from jax.experimental.pallas import tpu as pltpu
from jax.experimental.pallas import tpu_sc as plsc

N = 10000
E = 320000
DF = 128
H = 32
LAT = 16

NC = 2
NS = 16
NW = NC * NS
CH = 125
CHUNKS = E // CH
TCH = CHUNKS // NW
NB = 10
CP_TILES = 10
ROWS_PT = N // CP_TILES

_mesh = plsc.VectorSubcoreMesh(core_axis_name="c", subcore_axis_name="s")


@functools.partial(
    pl.kernel,
    out_type=jax.ShapeDtypeStruct((NC, N, H), jnp.float32),
    mesh=_mesh,
    compiler_params=pltpu.CompilerParams(use_tc_tiling_on_sc=False),
    scratch_types=[
        pltpu.VMEM_SHARED((N, H), jnp.float32),
        pltpu.VMEM((TCH, CH), jnp.int32),
        pltpu.VMEM((CH, H), jnp.float32),
        pltpu.SemaphoreType.DMA,
    ],
)
def _deg_kernel(dst2, ones, zeros32, out, deg_sh, idx_d, obuf, sem):
    cid = lax.axis_index("c")
    sid = lax.axis_index("s")
    wid = sid * NC + cid

    @pl.when(sid < CP_TILES)
    def _():
        r0 = sid * ROWS_PT
        pltpu.sync_copy(zeros32.at[pl.ds(r0, ROWS_PT)],
                        deg_sh.at[pl.ds(r0, ROWS_PT)])

    pltpu.sync_copy(dst2.at[pl.ds(wid * TCH, TCH)], idx_d)
    pltpu.sync_copy(ones, obuf)
    plsc.subcore_barrier()

    def step(j0, carry):
        descs = []
        for b in range(8):
            j = j0 * 8 + b
            descs.append(
                pltpu.async_copy(obuf, deg_sh.at[idx_d.at[j]], sem, add=True))
        for d in descs:
            d.wait()
        return carry

    lax.fori_loop(0, TCH // 8, step, 0)
    plsc.subcore_barrier()

    @pl.when(sid < CP_TILES)
    def _():
        r0 = sid * ROWS_PT
        pltpu.sync_copy(deg_sh.at[pl.ds(r0, ROWS_PT)],
                        out.at[cid, pl.ds(r0, ROWS_PT)])


@functools.partial(
    pl.kernel,
    out_type=jax.ShapeDtypeStruct((NC, N, H), jnp.float32),
    mesh=_mesh,
    compiler_params=pltpu.CompilerParams(use_tc_tiling_on_sc=False),
    scratch_types=[
        pltpu.VMEM_SHARED((N, H), jnp.float32),
        pltpu.VMEM((TCH, CH), jnp.int32),
        pltpu.VMEM((TCH, CH), jnp.int32),
        pltpu.VMEM((NB, CH, H), jnp.float32),
    ] + [pltpu.SemaphoreType.DMA] * NB,
)
def _prop_kernel(table, src2, dst2, zeros, out, acc_sh, idx_s, idx_d,
                 rows, *gsems):
    cid = lax.axis_index("c")
    sid = lax.axis_index("s")
    wid = sid * NC + cid

    @pl.when(sid < CP_TILES)
    def _():
        r0 = sid * ROWS_PT
        pltpu.sync_copy(zeros.at[pl.ds(r0, ROWS_PT)],
                        acc_sh.at[pl.ds(r0, ROWS_PT)])

    base = wid * TCH
    pltpu.sync_copy(src2.at[pl.ds(base, TCH)], idx_s)
    pltpu.sync_copy(dst2.at[pl.ds(base, TCH)], idx_d)
    plsc.subcore_barrier()

    for b in range(NB):
        pltpu.async_copy(table.at[idx_s.at[b]], rows.at[b], gsems[b])

    def step(j0, carry):
        for b in range(NB):
            j = j0 * NB + b
            pltpu.make_async_copy(table.at[idx_s.at[j]], rows.at[b],
                                  gsems[b]).wait()
            pltpu.sync_copy(rows.at[b], acc_sh.at[idx_d.at[j]], add=True)
            pltpu.async_copy(table.at[idx_s.at[j + NB]], rows.at[b], gsems[b])
        return carry

    lax.fori_loop(0, TCH // NB - 1, step, 0)

    for b in range(NB):
        j = TCH - NB + b
        pltpu.make_async_copy(table.at[idx_s.at[j]], rows.at[b],
                              gsems[b]).wait()
        pltpu.sync_copy(rows.at[b], acc_sh.at[idx_d.at[j]], add=True)

    plsc.subcore_barrier()

    @pl.when(sid < CP_TILES)
    def _():
        r0 = sid * ROWS_PT
        pltpu.sync_copy(acc_sh.at[pl.ds(r0, ROWS_PT)],
                        out.at[cid, pl.ds(r0, ROWS_PT)])


NW128 = N * H // 128
WX = 4 * DF


def _disw(degw_ref):
    return lax.rsqrt(degw_ref[0] + degw_ref[1] + 1.0)


def _tc1_body(xw_ref, w1k_ref, degw_ref, g2_ref):
    g = jnp.dot(xw_ref[...], w1k_ref[...], preferred_element_type=jnp.float32)
    g2_ref[...] = g * _disw(degw_ref)


_tc1 = pl.pallas_call(
    _tc1_body,
    grid=(1,),
    in_specs=[
        pl.BlockSpec((NW128, WX), lambda i: (0, 0)),
        pl.BlockSpec((WX, 128), lambda i: (0, 0)),
        pl.BlockSpec((NC, NW128, 128), lambda i: (0, 0, 0)),
    ],
    out_specs=pl.BlockSpec((NW128, 128), lambda i: (0, 0)),
    out_shape=jax.ShapeDtypeStruct((NW128, 128), jnp.float32),
)


def _tc2_body(g2_ref, acc_ref, degw_ref, b1t_ref, h2_ref):
    dis = _disw(degw_ref)
    z = dis * (g2_ref[...] + acc_ref[0] + acc_ref[1]) + b1t_ref[...]
    h2_ref[...] = dis * jnp.maximum(z, 0.0)


_tc2 = pl.pallas_call(
    _tc2_body,
    grid=(1,),
    in_specs=[
        pl.BlockSpec((NW128, 128), lambda i: (0, 0)),
        pl.BlockSpec((NC, NW128, 128), lambda i: (0, 0, 0)),
        pl.BlockSpec((NC, NW128, 128), lambda i: (0, 0, 0)),
        pl.BlockSpec((1, 128), lambda i: (0, 0)),
    ],
    out_specs=pl.BlockSpec((NW128, 128), lambda i: (0, 0)),
    out_shape=jax.ShapeDtypeStruct((NW128, 128), jnp.float32),
)


def _tc3_body(h2_ref, acc_ref, degw_ref, wmk_ref, bmt_ref, wvk_ref, bvt_ref,
              mean_ref, var_ref):
    p = _disw(degw_ref) * (h2_ref[...] + acc_ref[0] + acc_ref[1])
    mean_ref[...] = jnp.dot(p, wmk_ref[...],
                            preferred_element_type=jnp.float32) + bmt_ref[...]
    var_ref[...] = jnp.dot(p, wvk_ref[...],
                           preferred_element_type=jnp.float32) + bvt_ref[...]


_tc3 = pl.pallas_call(
    _tc3_body,
    grid=(1,),
    in_specs=[
        pl.BlockSpec((NW128, 128), lambda i: (0, 0)),
        pl.BlockSpec((NC, NW128, 128), lambda i: (0, 0, 0)),
        pl.BlockSpec((NC, NW128, 128), lambda i: (0, 0, 0)),
        pl.BlockSpec((128, 4 * LAT), lambda i: (0, 0)),
        pl.BlockSpec((1, 4 * LAT), lambda i: (0, 0)),
        pl.BlockSpec((128, 4 * LAT), lambda i: (0, 0)),
        pl.BlockSpec((1, 4 * LAT), lambda i: (0, 0)),
    ],
    out_specs=[
        pl.BlockSpec((NW128, 4 * LAT), lambda i: (0, 0)),
        pl.BlockSpec((NW128, 4 * LAT), lambda i: (0, 0)),
    ],
    out_shape=[
        jax.ShapeDtypeStruct((NW128, 4 * LAT), jnp.float32),
        jax.ShapeDtypeStruct((NW128, 4 * LAT), jnp.float32),
    ],
)


@jax.jit
def kernel(features, edge_index, W1, b1, Wm, bm, Wv, bv):
    src2 = edge_index[0].reshape(CHUNKS, CH)
    dst2 = edge_index[1].reshape(CHUNKS, CH)
    zeros = jnp.zeros((N, H), jnp.float32)
    ones = jnp.ones((CH, H), jnp.float32)
    eye4 = jnp.eye(4, dtype=jnp.float32)

    degp = _deg_kernel(dst2, ones, zeros)
    degw = degp.reshape(NC, NW128, 128)
    xw = features.reshape(NW128, WX)
    g2w = _tc1(xw, jnp.kron(eye4, W1), degw)
    g2 = g2w.reshape(N, H)
    acc1 = _prop_kernel(g2, src2, dst2, zeros)
    h2w = _tc2(g2w, acc1.reshape(NC, NW128, 128), degw,
               jnp.tile(b1, 4).reshape(1, 128))
    h2 = h2w.reshape(N, H)
    acc2 = _prop_kernel(h2, src2, dst2, zeros)
    meanw, varw = _tc3(h2w, acc2.reshape(NC, NW128, 128), degw,
                       jnp.kron(eye4, Wm), jnp.tile(bm, 4).reshape(1, 64),
                       jnp.kron(eye4, Wv), jnp.tile(bv, 4).reshape(1, 64))
    return (meanw.reshape(N, LAT), varw.reshape(N, LAT))
